# baseline (speedup 1.0000x reference)
import jax
import jax.numpy as jnp
from jax import lax
from jax.experimental import pallas as pl
from jax.experimental.pallas import tpu as pltpu
from jax.experimental.pallas import tpu_sc as plsc

NUM_CORES = 2
NUM_SUBCORES = 16
LANES = 16
NW = NUM_CORES * NUM_SUBCORES
BATCH_SIZE = 16384
DIM = 64
PAIR = 128
NENT = 1000000
NPAIR = NENT // 2
NCOL = 999936 // PAIR
COLS_PER_W = -(-NCOL // NW)
RELROWS = 1024
BPW = BATCH_SIZE // NW
CHUNK = 128
NCHUNK = BPW // CHUNK
BLOCKS_PER_CHUNK = CHUNK // LANES


def _fmt_body(et_hbm, rt_hbm, tail_hbm, ent_out, rel_out, it, ot, sem_i, sem_o):
    wid = lax.axis_index("s") * NUM_CORES + lax.axis_index("c")

    iota16 = lax.iota(jnp.int32, LANES)
    dvecs = [a * LANES + iota16 for a in range(4)]

    def col_of(i):
        return lax.min(wid + i * NW, NCOL - 1)

    def fire_in(i, half):
        src = pl.ds(pl.multiple_of(col_of(i) * PAIR, PAIR), PAIR)
        for dt in range(8):
            rows = pl.ds(dt * 8, 8)
            pltpu.async_copy(et_hbm.at[rows, src], it.at[half, rows], sem_i)

    def drain_in(half):
        for dt in range(8):
            rows = pl.ds(dt * 8, 8)
            pltpu.make_async_copy(et_hbm.at[rows, pl.ds(0, PAIR)],
                                  it.at[half, rows], sem_i).wait()

    def fire_out(i, half):
        dst = pl.ds(col_of(i) * DIM, DIM)
        pltpu.async_copy(ot.at[half], ent_out.at[dst], sem_o)

    def drain_out(half):
        pltpu.make_async_copy(ot.at[half], ent_out.at[pl.ds(0, DIM)],
                              sem_o).wait()

    def transpose(src_ref, dst_ref):
        @plsc.parallel_loop(0, 8, unroll=2)
        def prow(it8):
            for pj in range(8):
                pbase = it8 * LANES + 2 * pj
                c0 = jnp.full((LANES,), pbase, jnp.int32)
                c1 = jnp.full((LANES,), pbase + 1, jnp.int32)
                for a in range(4):
                    v0 = plsc.load_gather(src_ref, [dvecs[a], c0])
                    v1 = plsc.load_gather(src_ref, [dvecs[a], c1])
                    row = it8 * 8 + pj
                    dst_ref[row, pl.ds(a * LANES, LANES)] = v0
                    dst_ref[row, pl.ds(DIM + a * LANES, LANES)] = v1

    fire_in(0, 0)
    fire_in(1, 1)

    def body(g, carry):
        for half in range(2):
            i = g * 2 + half
            drain_in(half)

            @pl.when(i >= 2)
            def _():
                drain_out(half)

            transpose(it.at[half], ot.at[half])
            fire_out(i, half)

            @pl.when(i + 2 < COLS_PER_W)
            def _():
                fire_in(i + 2, half)
        return carry

    lax.fori_loop(0, COLS_PER_W // 2, body, 0)
    i = COLS_PER_W - 1
    drain_in(i % 2)
    drain_out(i % 2)
    transpose(it.at[i % 2], ot.at[i % 2])
    fire_out(i, i % 2)
    drain_out((i + 1) % 2)
    drain_out(i % 2)

    @pl.when(wid < RELROWS // PAIR)
    def _():
        src = pl.ds(pl.multiple_of(wid * PAIR, PAIR), PAIR)
        pltpu.async_copy(rt_hbm.at[:, src], it.at[0], sem_i).wait()
        transpose(it.at[0], ot.at[0])
        pltpu.async_copy(ot.at[0], rel_out.at[pl.ds(wid * DIM, DIM)],
                         sem_o).wait()

    @pl.when(wid == NW - 1)
    def _():
        pltpu.async_copy(tail_hbm, it.at[1], sem_i).wait()
        transpose(it.at[1], ot.at[1])
        pltpu.async_copy(ot.at[1, pl.ds(0, 32)],
                         ent_out.at[pl.ds(NCOL * DIM, 32)], sem_o).wait()


_fmt_call = pl.kernel(
    _fmt_body,
    out_type=(jax.ShapeDtypeStruct((NPAIR, PAIR), jnp.float32),
              jax.ShapeDtypeStruct((RELROWS // 2, PAIR), jnp.float32)),
    mesh=plsc.VectorSubcoreMesh(core_axis_name="c", subcore_axis_name="s"),
    compiler_params=pltpu.CompilerParams(needs_layout_passes=False,
                                         use_tc_tiling_on_sc=True),
    scratch_types=[
        pltpu.VMEM((2, DIM, PAIR), jnp.float32),
        pltpu.VMEM((2, DIM, PAIR), jnp.float32),
        pltpu.SemaphoreType.DMA,
        pltpu.SemaphoreType.DMA,
    ],
)


def _tec_body(h_hbm, r_hbm, t_hbm, ow_hbm, ent_hbm, rel_hbm, out_hbm,
              hi, ri, ti, hv, rv, tv, ov, ps, ows, sem):
    wid = lax.axis_index("s") * NUM_CORES + lax.axis_index("c")
    base = wid * BPW

    pltpu.sync_copy(h_hbm.at[wid], hi)
    pltpu.sync_copy(r_hbm.at[wid], ri)
    pltpu.sync_copy(t_hbm.at[wid], ti)
    pltpu.sync_copy(ow_hbm.at[wid], ows)

    iota16 = lax.iota(jnp.int32, LANES)
    rowbase = iota16 * LANES

    def fire(c):
        half = c % 2
        pltpu.async_copy(ent_hbm.at[hi.at[c]], hv.at[half], sem)
        pltpu.async_copy(ent_hbm.at[ti.at[c]], tv.at[half], sem)
        pltpu.async_copy(rel_hbm.at[ri.at[c]], rv.at[half], sem)

    def drain(c):
        half = c % 2
        pltpu.make_async_copy(ent_hbm.at[hi.at[c]], hv.at[half], sem).wait()
        pltpu.make_async_copy(ent_hbm.at[ti.at[c]], tv.at[half], sem).wait()
        pltpu.make_async_copy(rel_hbm.at[ri.at[c]], rv.at[half], sem).wait()

    def make_block(c):
        half = c % 2

        def block(blk, carry):
            owsv = ows[pl.ds(c * CHUNK + blk * LANES, LANES)]
            for j in range(LANES):
                row = blk * LANES + j
                ow = owsv[j]
                hoff = lax.shift_left(lax.bitwise_and(ow, 1), 6)
                toff = lax.shift_left(lax.bitwise_and(ow, 2), 5)
                roff = lax.shift_left(lax.bitwise_and(ow, 4), 4)
                sq = []
                for a in range(4):
                    hx = hv.at[half, row][pl.ds(hoff + a * LANES, LANES)]
                    tx = tv.at[half, row][pl.ds(toff + a * LANES, LANES)]
                    rx = rv.at[half, row][pl.ds(roff + a * LANES, LANES)]
                    df = (hx - tx) + rx
                    sq.append(df * df)
                ps[pl.ds(j * LANES, LANES)] = (sq[0] + sq[1]) + (sq[2] + sq[3])

            acc = plsc.load_gather(ps, [rowbase])
            for l in range(1, LANES):
                acc = acc + plsc.load_gather(ps, [rowbase + l])

            bits = plsc.bitcast(acc, jnp.int32)
            y = plsc.bitcast(
                jnp.int32(0x5F3759DF) - lax.shift_right_logical(bits, 1),
                jnp.float32)
            for _ in range(3):
                y = y * (1.5 - 0.5 * acc * y * y)
            ov[pl.ds(c * CHUNK + blk * LANES, LANES)] = acc * y
            return carry

        return block

    fire(0)
    fire(1)
    for c in range(NCHUNK):
        drain(c)
        lax.fori_loop(0, BLOCKS_PER_CHUNK, make_block(c), 0)
        if c + 2 < NCHUNK:
            fire(c + 2)

    pltpu.sync_copy(ov, out_hbm.at[pl.ds(base, BPW)])


_sc_call = pl.kernel(
    _tec_body,
    out_type=jax.ShapeDtypeStruct((BATCH_SIZE,), jnp.float32),
    mesh=plsc.VectorSubcoreMesh(core_axis_name="c", subcore_axis_name="s"),
    compiler_params=pltpu.CompilerParams(needs_layout_passes=False,
                                         use_tc_tiling_on_sc=True),
    scratch_types=[
        pltpu.VMEM((NCHUNK, CHUNK), jnp.int32),
        pltpu.VMEM((NCHUNK, CHUNK), jnp.int32),
        pltpu.VMEM((NCHUNK, CHUNK), jnp.int32),
        pltpu.VMEM((2, CHUNK, PAIR), jnp.float32),
        pltpu.VMEM((2, CHUNK, PAIR), jnp.float32),
        pltpu.VMEM((2, CHUNK, PAIR), jnp.float32),
        pltpu.VMEM((BPW,), jnp.float32),
        pltpu.VMEM((LANES * LANES,), jnp.float32),
        pltpu.VMEM((BPW,), jnp.int32),
        pltpu.SemaphoreType.DMA,
    ],
)


def kernel(h, r, t, entity_embedding, relation_embedding):
    h = h.astype(jnp.int32)
    r = r.astype(jnp.int32)
    t = t.astype(jnp.int32)
    hp = lax.shift_right_logical(h, 1).reshape(NW, NCHUNK, CHUNK)
    rp = lax.shift_right_logical(r, 1).reshape(NW, NCHUNK, CHUNK)
    tp = lax.shift_right_logical(t, 1).reshape(NW, NCHUNK, CHUNK)
    ow = ((h & 1) | ((t & 1) << 1) | ((r & 1) << 2)).reshape(NW, BPW)
    relp = jnp.pad(relation_embedding, ((0, RELROWS - 1000), (0, 0)))
    tail = jnp.pad(entity_embedding[NCOL * PAIR:].T, ((0, 0), (0, DIM)))
    ent2, rel2 = _fmt_call(entity_embedding.T, relp.T, tail)
    return _sc_call(hp, rp, tp, ow, ent2, rel2)


# speedup vs baseline: 1.5698x; 1.5698x over previous
import jax
import jax.numpy as jnp
from jax import lax
from jax.experimental import pallas as pl
from jax.experimental.pallas import tpu as pltpu
from jax.experimental.pallas import tpu_sc as plsc

NUM_CORES = 2
NUM_SUBCORES = 16
LANES = 16
NW = NUM_CORES * NUM_SUBCORES
BATCH_SIZE = 16384
DIM = 64
PADW = 128
BPW = BATCH_SIZE // NW
CHUNK = 128
NCHUNK = BPW // CHUNK
BLOCKS_PER_CHUNK = CHUNK // LANES


def _tec_body(h_hbm, r_hbm, t_hbm, ent_hbm, rel_hbm, out_hbm,
              hi, ri, ti, hv, rv, tv, ov, ps, sem):
    wid = lax.axis_index("s") * NUM_CORES + lax.axis_index("c")
    base = wid * BPW

    pltpu.sync_copy(h_hbm.at[wid], hi)
    pltpu.sync_copy(r_hbm.at[wid], ri)
    pltpu.sync_copy(t_hbm.at[wid], ti)

    iota16 = lax.iota(jnp.int32, LANES)
    rowbase = iota16 * LANES

    def fire(c):
        half = c % 2
        pltpu.async_copy(ent_hbm.at[hi.at[c]], hv.at[half], sem)
        pltpu.async_copy(ent_hbm.at[ti.at[c]], tv.at[half], sem)
        pltpu.async_copy(rel_hbm.at[ri.at[c]], rv.at[half], sem)

    def drain(c):
        half = c % 2
        pltpu.make_async_copy(ent_hbm.at[hi.at[c]], hv.at[half], sem).wait()
        pltpu.make_async_copy(ent_hbm.at[ti.at[c]], tv.at[half], sem).wait()
        pltpu.make_async_copy(rel_hbm.at[ri.at[c]], rv.at[half], sem).wait()

    def make_block(c):
        half = c % 2

        def block(blk, carry):
            for j in range(LANES):
                row = blk * LANES + j
                sq = []
                for a in range(4):
                    sl = pl.ds(a * LANES, LANES)
                    df = ((hv.at[half, row][sl] - tv.at[half, row][sl])
                          + rv.at[half, row][sl])
                    sq.append(df * df)
                ps[pl.ds(j * LANES, LANES)] = (sq[0] + sq[1]) + (sq[2] + sq[3])

            acc = plsc.load_gather(ps, [rowbase])
            for l in range(1, LANES):
                acc = acc + plsc.load_gather(ps, [rowbase + l])

            bits = plsc.bitcast(acc, jnp.int32)
            y = plsc.bitcast(
                jnp.int32(0x5F3759DF) - lax.shift_right_logical(bits, 1),
                jnp.float32)
            for _ in range(3):
                y = y * (1.5 - 0.5 * acc * y * y)
            ov[pl.ds(c * CHUNK + blk * LANES, LANES)] = acc * y
            return carry

        return block

    fire(0)
    fire(1)
    for c in range(NCHUNK):
        drain(c)
        lax.fori_loop(0, BLOCKS_PER_CHUNK, make_block(c), 0)
        if c + 2 < NCHUNK:
            fire(c + 2)

    pltpu.sync_copy(ov, out_hbm.at[pl.ds(base, BPW)])


_sc_call = pl.kernel(
    _tec_body,
    out_type=jax.ShapeDtypeStruct((BATCH_SIZE,), jnp.float32),
    mesh=plsc.VectorSubcoreMesh(core_axis_name="c", subcore_axis_name="s"),
    compiler_params=pltpu.CompilerParams(needs_layout_passes=False,
                                         use_tc_tiling_on_sc=False),
    scratch_types=[
        pltpu.VMEM((NCHUNK, CHUNK), jnp.int32),
        pltpu.VMEM((NCHUNK, CHUNK), jnp.int32),
        pltpu.VMEM((NCHUNK, CHUNK), jnp.int32),
        pltpu.VMEM((2, CHUNK, PADW), jnp.float32),
        pltpu.VMEM((2, CHUNK, PADW), jnp.float32),
        pltpu.VMEM((2, CHUNK, PADW), jnp.float32),
        pltpu.VMEM((BPW,), jnp.float32),
        pltpu.VMEM((LANES * LANES,), jnp.float32),
        pltpu.SemaphoreType.DMA,
    ],
)


def kernel(h, r, t, entity_embedding, relation_embedding):
    h3 = h.astype(jnp.int32).reshape(NW, NCHUNK, CHUNK)
    r3 = r.astype(jnp.int32).reshape(NW, NCHUNK, CHUNK)
    t3 = t.astype(jnp.int32).reshape(NW, NCHUNK, CHUNK)
    ent2 = jnp.pad(entity_embedding, ((0, 0), (0, PADW - DIM)))
    rel2 = jnp.pad(relation_embedding, ((0, 0), (0, PADW - DIM)))
    return _sc_call(h3, r3, t3, ent2, rel2)
